# baseline (speedup 1.0000x reference)
import functools

import jax
import jax.numpy as jnp
from jax import lax
from jax.experimental import pallas as pl
from jax.experimental.pallas import tpu as pltpu
from jax.experimental.pallas import tpu_sc as plsc

NY, NX, C = 496, 432, 64
M = NY * NX
N_VOX = 120000
NW = 32
BLK = 128
NBPW = 53
RPW = NBPW * BLK
MPAD = NW * RPW
NZ = 1024
CHUNK = 6000
NCHUNK = N_VOX // CHUNK


def _sc_body(idx_hbm, feat_hbm, outT_hbm, owner, idxb0, idxb1, rows0, rows1,
             si0, si1, sg0, sg1, sw0, sw1):
    idxb = (idxb0, idxb1)
    rows = (rows0, rows1)
    wid = lax.axis_index("s") * 2 + lax.axis_index("c")
    lane = lax.iota(jnp.int32, 16)
    lo = wid * RPW

    def initb(t, carry):
        owner[pl.ds(t * 16, 16)] = N_VOX + ((t * 16 + lane) & (NZ - 1))
        return carry
    lax.fori_loop(0, RPW // 16, initb, 0)

    sems_i = (si0, si1)
    hs = [pltpu.async_copy(idx_hbm.at[pl.ds(0, CHUNK)], idxb[0], si0), None]
    for k in range(NCHUNK):
        p = k & 1
        if k + 1 < NCHUNK:
            hs[1 - p] = pltpu.async_copy(
                idx_hbm.at[pl.ds((k + 1) * CHUNK, CHUNK)],
                idxb[1 - p], sems_i[1 - p])
        hs[p].wait()

        def scan_t(t, carry, p=p, kbase=k * CHUNK):
            idx16 = idxb[p][pl.ds(t * 16, 16)]
            ivec = kbase + t * 16 + lane
            loc = idx16 - lo
            inr = plsc.bitcast(loc, jnp.uint32) < jnp.uint32(RPW)
            plsc.store_scatter(owner, [loc], ivec, mask=inr)
            cnt = plsc.all_reduce_population_count(inr)[0]

            @pl.when(cnt >= 2)
            def _fixup():
                rb = plsc.load_gather(owner, [loc], mask=inr)
                need0 = inr & (rb < ivec)

                def wcond(need):
                    return plsc.all_reduce_population_count(need)[0] > 0

                def wbody(need):
                    plsc.store_scatter(owner, [loc], ivec, mask=need)
                    rb2 = plsc.load_gather(owner, [loc], mask=inr)
                    return inr & (rb2 < ivec)

                lax.while_loop(wcond, wbody, need0)
            return carry
        lax.fori_loop(0, CHUNK // 16, scan_t, 0)

    sems_g = (sg0, sg1)
    sems_w = (sw0, sw1)
    gh = [pltpu.async_copy(
        feat_hbm.at[owner.at[pl.ds(0, BLK)]], rows[0], sg0), None]
    wh = [None, None]
    for b in range(NBPW):
        p = b & 1
        q = 1 - p
        if b + 1 < NBPW:
            if wh[q] is not None:
                wh[q].wait()
            gh[q] = pltpu.async_copy(
                feat_hbm.at[owner.at[pl.ds((b + 1) * BLK, BLK)]],
                rows[q], sems_g[q])
        gh[p].wait()
        wh[p] = pltpu.async_copy(
            rows[p], outT_hbm.at[pl.ds((wid * NBPW + b) * BLK, BLK)],
            sems_w[p])
    wh[0].wait()
    wh[1].wait()


_sc_scatter = functools.partial(
    pl.kernel,
    out_type=jax.ShapeDtypeStruct((MPAD, C), jnp.float32),
    mesh=plsc.VectorSubcoreMesh(core_axis_name="c", subcore_axis_name="s"),
    compiler_params=pltpu.CompilerParams(
        needs_layout_passes=False, use_tc_tiling_on_sc=False),
    scratch_types=[
        pltpu.VMEM((RPW,), jnp.int32),
        pltpu.VMEM((CHUNK,), jnp.int32),
        pltpu.VMEM((CHUNK,), jnp.int32),
        pltpu.VMEM((BLK, C), jnp.float32),
        pltpu.VMEM((BLK, C), jnp.float32),
        pltpu.SemaphoreType.DMA,
        pltpu.SemaphoreType.DMA,
        pltpu.SemaphoreType.DMA,
        pltpu.SemaphoreType.DMA,
        pltpu.SemaphoreType.DMA,
        pltpu.SemaphoreType.DMA,
    ],
)(_sc_body)


TRB = 256


def _tr_body(x_ref, o_ref):
    o_ref[...] = x_ref[...].T


_transpose = pl.pallas_call(
    _tr_body,
    grid=(M // TRB,),
    in_specs=[pl.BlockSpec((TRB, C), lambda i: (i, 0))],
    out_specs=pl.BlockSpec((C, TRB), lambda i: (0, i)),
    out_shape=jax.ShapeDtypeStruct((C, M), jnp.float32),
)


def kernel(voxel_features, coors):
    idx = coors[:, 1] * NX + coors[:, 2]
    feat_ext = jnp.concatenate(
        [voxel_features, jnp.zeros((NZ, C), jnp.float32)], axis=0)
    canvas_t = _sc_scatter(idx, feat_ext)
    canvas = _transpose(canvas_t)
    return (jnp.reshape(canvas, (1, C, NY, NX)),)
